# baseline (speedup 1.0000x reference)
import jax
import jax.numpy as jnp
from jax import lax
from jax.experimental import pallas as pl
from jax.experimental.pallas import tpu as pltpu
from jax.experimental.pallas import tpu_sc as plsc

_PARENT = (1, 20, 20, 2, 20, 4, 5, 6, 20, 8, 9, 10, 0, 12, 13, 14, 0, 16,
           17, 18, 20, 22, 7, 24, 11)

_B, _C, _V, _T = 512, 3, 25, 300
_ROWS = _B * _C
_GROUP = 2
_NGROUPS = _ROWS // _GROUP
_GWORDS = _GROUP * _V * _T
_NW = 32
_GPW = _NGROUPS // _NW

_FULL = _T // 16
_TAIL = _T - 16


def _sc_body(x_hbm, out_hbm, xbuf, obuf):
    wid = lax.axis_index("s") * 2 + lax.axis_index("c")
    base = wid * _GPW

    def do_group(g, carry):
        row = base + g
        pltpu.sync_copy(x_hbm.at[row], xbuf)

        def do_chunk(k, c2):
            off = jnp.minimum(k * 16, _TAIL)
            for s in range(_GROUP):
                sb = s * _V * _T
                regs = [xbuf[pl.ds(sb + j * _T + off, 16)] for j in range(_V)]
                for j in range(_V):
                    obuf[pl.ds(sb + j * _T + off, 16)] = (
                        regs[j] - regs[_PARENT[j]])
            return c2

        lax.fori_loop(0, _FULL + 1, do_chunk, 0)
        pltpu.sync_copy(obuf, out_hbm.at[row])
        return carry

    lax.fori_loop(0, _GPW, do_group, 0)


def kernel(joint_data):
    x = joint_data.reshape(_NGROUPS, _GWORDS)
    mesh = plsc.VectorSubcoreMesh(core_axis_name="c", subcore_axis_name="s")
    f = pl.kernel(
        _sc_body,
        mesh=mesh,
        out_type=jax.ShapeDtypeStruct((_NGROUPS, _GWORDS), jnp.float32),
        scratch_types=[
            pltpu.VMEM((_GWORDS,), jnp.float32),
            pltpu.VMEM((_GWORDS,), jnp.float32),
        ],
    )
    out = f(x)
    return out.reshape(_B, _C, _V, _T)


# speedup vs baseline: 1.1072x; 1.1072x over previous
import jax
import jax.numpy as jnp
from jax import lax
from jax.experimental import pallas as pl
from jax.experimental.pallas import tpu as pltpu
from jax.experimental.pallas import tpu_sc as plsc

_PARENT = (1, 20, 20, 2, 20, 4, 5, 6, 20, 8, 9, 10, 0, 12, 13, 14, 0, 16,
           17, 18, 20, 22, 7, 24, 11)

_B, _C, _V, _T = 512, 3, 25, 300
_ROWS = _B * _C
_GROUP = 4
_NGROUPS = _ROWS // _GROUP
_GWORDS = _GROUP * _V * _T
_NW = 32
_GPW = _NGROUPS // _NW

_FULL = _T // 16
_TAIL = _T - 16


def _compute(xbuf, obuf):
    def do_chunk(k, c2):
        off = jnp.minimum(k * 16, _TAIL)
        for s in range(_GROUP):
            sb = s * _V * _T
            regs = [xbuf[pl.ds(sb + j * _T + off, 16)] for j in range(_V)]
            for j in range(_V):
                obuf[pl.ds(sb + j * _T + off, 16)] = (
                    regs[j] - regs[_PARENT[j]])
        return c2

    lax.fori_loop(0, _FULL + 1, do_chunk, 0)


def _sc_body(x_hbm, out_hbm, xb0, xb1, ob0, ob1, si0, si1, so0, so1):
    wid = lax.axis_index("s") * 2 + lax.axis_index("c")
    base = wid * _GPW
    xbufs, obufs = (xb0, xb1), (ob0, ob1)
    sins, souts = (si0, si1), (so0, so1)

    pltpu.make_async_copy(x_hbm.at[base], xbufs[0], sins[0]).start()

    def do_pair(gp, carry):
        for b in range(2):
            g = gp * 2 + b
            @pl.when(g + 1 < _GPW)
            def _():
                pltpu.make_async_copy(
                    x_hbm.at[base + g + 1], xbufs[1 - b], sins[1 - b]).start()

            pltpu.make_async_copy(
                x_hbm.at[base + g], xbufs[b], sins[b]).wait()

            @pl.when(g >= 2)
            def _():
                pltpu.make_async_copy(
                    obufs[b], out_hbm.at[base + g - 2], souts[b]).wait()

            _compute(xbufs[b], obufs[b])
            pltpu.make_async_copy(
                obufs[b], out_hbm.at[base + g], souts[b]).start()
        return carry

    lax.fori_loop(0, _GPW // 2, do_pair, 0)

    for b in range(2):
        pltpu.make_async_copy(
            obufs[b], out_hbm.at[base + _GPW - 2 + b], souts[b]).wait()


def kernel(joint_data):
    x = joint_data.reshape(_NGROUPS, _GWORDS)
    mesh = plsc.VectorSubcoreMesh(core_axis_name="c", subcore_axis_name="s")
    f = pl.kernel(
        _sc_body,
        mesh=mesh,
        out_type=jax.ShapeDtypeStruct((_NGROUPS, _GWORDS), jnp.float32),
        scratch_types=[
            pltpu.VMEM((_GWORDS,), jnp.float32),
            pltpu.VMEM((_GWORDS,), jnp.float32),
            pltpu.VMEM((_GWORDS,), jnp.float32),
            pltpu.VMEM((_GWORDS,), jnp.float32),
            pltpu.SemaphoreType.DMA,
            pltpu.SemaphoreType.DMA,
            pltpu.SemaphoreType.DMA,
            pltpu.SemaphoreType.DMA,
        ],
    )
    out = f(x)
    return out.reshape(_B, _C, _V, _T)


# speedup vs baseline: 1.5892x; 1.4353x over previous
import jax
import jax.numpy as jnp
from jax import lax
from jax.experimental import pallas as pl
from jax.experimental.pallas import tpu as pltpu
from jax.experimental.pallas import tpu_sc as plsc

_PARENT = (1, 20, 20, 2, 20, 4, 5, 6, 20, 8, 9, 10, 0, 12, 13, 14, 0, 16,
           17, 18, 20, 22, 7, 24, 11)

_B, _C, _V, _T = 512, 3, 25, 300
_UNITS = _B * _C
_NW = 32
_UPW = _UNITS // _NW

_FULL = _T // 16
_TAIL = _T - 16


def _compute(xbuf, obuf):
    def do_chunk(k, c2):
        off = pl.multiple_of(k * 16, 16)
        regs = [xbuf[j, pl.ds(off, 16)] for j in range(_V)]
        for j in range(_V):
            obuf[j, pl.ds(off, 16)] = regs[j] - regs[_PARENT[j]]
        return c2

    lax.fori_loop(0, _FULL, do_chunk, 0)

    regs = [xbuf[j, pl.ds(_TAIL, 16)] for j in range(_V)]
    for j in range(_V):
        obuf[j, pl.ds(_TAIL, 16)] = regs[j] - regs[_PARENT[j]]


def _sc_body(x_hbm, out_hbm, xb0, xb1, ob0, ob1, si0, si1, so0, so1):
    wid = lax.axis_index("s") * 2 + lax.axis_index("c")
    base = wid * _UPW
    xbufs, obufs = (xb0, xb1), (ob0, ob1)
    sins, souts = (si0, si1), (so0, so1)

    def src(u):
        return x_hbm.at[u // _C, u % _C]

    def dst(u):
        return out_hbm.at[u // _C, u % _C]

    pltpu.make_async_copy(src(base), xbufs[0], sins[0]).start()

    def do_pair(gp, carry):
        for b in range(2):
            g = gp * 2 + b
            @pl.when(g + 1 < _UPW)
            def _():
                pltpu.make_async_copy(
                    src(base + g + 1), xbufs[1 - b], sins[1 - b]).start()

            pltpu.make_async_copy(src(base + g), xbufs[b], sins[b]).wait()

            @pl.when(g >= 2)
            def _():
                pltpu.make_async_copy(
                    obufs[b], dst(base + g - 2), souts[b]).wait()

            _compute(xbufs[b], obufs[b])
            pltpu.make_async_copy(obufs[b], dst(base + g), souts[b]).start()
        return carry

    lax.fori_loop(0, _UPW // 2, do_pair, 0)

    for b in range(2):
        pltpu.make_async_copy(
            obufs[b], dst(base + _UPW - 2 + b), souts[b]).wait()


def kernel(joint_data):
    mesh = plsc.VectorSubcoreMesh(core_axis_name="c", subcore_axis_name="s")
    f = pl.kernel(
        _sc_body,
        mesh=mesh,
        out_type=jax.ShapeDtypeStruct((_B, _C, _V, _T), jnp.float32),
        scratch_types=[
            pltpu.VMEM((_V, _T), jnp.float32),
            pltpu.VMEM((_V, _T), jnp.float32),
            pltpu.VMEM((_V, _T), jnp.float32),
            pltpu.VMEM((_V, _T), jnp.float32),
            pltpu.SemaphoreType.DMA,
            pltpu.SemaphoreType.DMA,
            pltpu.SemaphoreType.DMA,
            pltpu.SemaphoreType.DMA,
        ],
    )
    return f(joint_data)


# speedup vs baseline: 5.3992x; 3.3974x over previous
import jax
import jax.numpy as jnp
from jax import lax
from jax.experimental import pallas as pl
from jax.experimental.pallas import tpu as pltpu
from jax.experimental.pallas import tpu_sc as plsc

_PARENT = (1, 20, 20, 2, 20, 4, 5, 6, 20, 8, 9, 10, 0, 12, 13, 14, 0, 16,
           17, 18, 20, 22, 7, 24, 11)

_B, _C, _V, _T = 512, 3, 25, 300
_UNITS = _C * _T
_NW = 32
_Q, _R = divmod(_UNITS, _NW)
_MAXU = _Q + 1

_CHUNKS = _B // 16


def _compute(xbuf, obuf):
    def do_chunk(k, c2):
        off = pl.multiple_of(k * 16, 16)
        regs = [xbuf[j, pl.ds(off, 16)] for j in range(_V)]
        for j in range(_V):
            obuf[j, pl.ds(off, 16)] = regs[j] - regs[_PARENT[j]]
        return c2

    lax.fori_loop(0, _CHUNKS, do_chunk, 0)


def _sc_body(x_hbm, out_hbm, xb0, xb1, ob0, ob1, si0, si1, so0, so1):
    wid = lax.axis_index("s") * 2 + lax.axis_index("c")
    base = wid * _Q + jnp.minimum(wid, _R)
    cnt = _Q + (wid < _R).astype(jnp.int32)
    xbufs, obufs = (xb0, xb1), (ob0, ob1)
    sins, souts = (si0, si1), (so0, so1)

    def src(i):
        u = base + i
        return x_hbm.at[u // _T, :, u % _T]

    def dst(i):
        u = base + i
        return out_hbm.at[u // _T, :, u % _T]

    pltpu.make_async_copy(src(0), xbufs[0], sins[0]).start()

    def do_pair(gp, carry):
        for b in range(2):
            i = gp * 2 + b
            @pl.when(i + 1 < cnt)
            def _():
                pltpu.make_async_copy(
                    src(i + 1), xbufs[1 - b], sins[1 - b]).start()

            @pl.when(i < cnt)
            def _():
                pltpu.make_async_copy(src(i), xbufs[b], sins[b]).wait()

            @pl.when(jnp.logical_and(i >= 2, i < cnt))
            def _():
                pltpu.make_async_copy(obufs[b], dst(i - 2), souts[b]).wait()

            @pl.when(i < cnt)
            def _():
                _compute(xbufs[b], obufs[b])
                pltpu.make_async_copy(obufs[b], dst(i), souts[b]).start()
        return carry

    lax.fori_loop(0, (_MAXU + 1) // 2, do_pair, 0)

    for b in range(2):
        pltpu.make_async_copy(obufs[b], dst(cnt - 2 + b), souts[b]).wait()


def kernel(joint_data):
    x = jnp.transpose(joint_data, (1, 2, 3, 0))
    mesh = plsc.VectorSubcoreMesh(core_axis_name="c", subcore_axis_name="s")
    f = pl.kernel(
        _sc_body,
        mesh=mesh,
        out_type=jax.ShapeDtypeStruct((_C, _V, _T, _B), jnp.float32),
        scratch_types=[
            pltpu.VMEM((_V, _B), jnp.float32),
            pltpu.VMEM((_V, _B), jnp.float32),
            pltpu.VMEM((_V, _B), jnp.float32),
            pltpu.VMEM((_V, _B), jnp.float32),
            pltpu.SemaphoreType.DMA,
            pltpu.SemaphoreType.DMA,
            pltpu.SemaphoreType.DMA,
            pltpu.SemaphoreType.DMA,
        ],
    )
    out = f(x)
    return jnp.transpose(out, (3, 0, 1, 2))
